# baseline (speedup 1.0000x reference)
import functools

import jax
import jax.numpy as jnp
from jax import lax
from jax.experimental import pallas as pl
from jax.experimental.pallas import tpu as pltpu
from jax.experimental.pallas import tpu_sc as plsc

N = 10000
E = 320000
D_IN = 128
D_HID = 5
D_OUT = 40

NC = 2
NS = 16
NW = NC * NS
EPW = E // NW
L = 16

_mesh = plsc.VectorSubcoreMesh(
    core_axis_name="c", subcore_axis_name="s", num_cores=NC, num_subcores=NS
)

_sc_params = pltpu.CompilerParams(needs_layout_passes=False)


def _worker_id():
    return lax.axis_index("s") * NC + lax.axis_index("c")


@functools.partial(
    pl.kernel,
    out_type=jax.ShapeDtypeStruct((NW, N), jnp.float32),
    mesh=_mesh,
    scratch_types=[
        pltpu.VMEM((EPW,), jnp.int32),
        pltpu.VMEM((N,), jnp.float32),
        pltpu.SemaphoreType.DMA,
    ],
    compiler_params=_sc_params,
)
def _deg_kernel(ei_hbm, out_hbm, dst_v, acc_v, sem):
    wid = _worker_id()
    base = wid * EPW
    cp = pltpu.async_copy(ei_hbm.at[pl.ds(E + base, EPW)], dst_v, sem)

    zeros = jnp.zeros((L,), jnp.float32)

    def zbody(i, _):
        acc_v[pl.ds(i * L, L)] = zeros
        return 0

    lax.fori_loop(0, N // L, zbody, 0, unroll=25)
    cp.wait()

    ones = jnp.ones((L,), jnp.float32)

    @plsc.parallel_loop(0, EPW // L, unroll=8)
    def _(i):
        d = dst_v[pl.ds(i * L, L)]
        plsc.addupdate_scatter(acc_v, [d], ones)

    pltpu.sync_copy(acc_v, out_hbm.at[wid])


@functools.partial(
    pl.kernel,
    out_type=jax.ShapeDtypeStruct((NW, D_HID * N), jnp.float32),
    mesh=_mesh,
    scratch_types=[
        pltpu.VMEM((D_HID * N,), jnp.float32),
        pltpu.VMEM((EPW,), jnp.int32),
        pltpu.VMEM((EPW,), jnp.int32),
        pltpu.VMEM((D_HID * N,), jnp.float32),
        pltpu.SemaphoreType.DMA,
        pltpu.SemaphoreType.DMA,
        pltpu.SemaphoreType.DMA,
    ],
    compiler_params=_sc_params,
)
def _scatter_kernel(
    g_hbm, ei_hbm, out_hbm, g_v, src_v, dst_v, acc_v, sem_g, sem_s, sem_d
):
    wid = _worker_id()
    base = wid * EPW
    cp_g = pltpu.async_copy(g_hbm.at[0], g_v, sem_g)
    cp_s = pltpu.async_copy(ei_hbm.at[pl.ds(base, EPW)], src_v, sem_s)
    cp_d = pltpu.async_copy(ei_hbm.at[pl.ds(E + base, EPW)], dst_v, sem_d)

    zeros = jnp.zeros((L,), jnp.float32)

    def zbody(i, _):
        acc_v[pl.ds(i * L, L)] = zeros
        return 0

    lax.fori_loop(0, D_HID * N // L, zbody, 0, unroll=25)
    cp_g.wait()
    cp_s.wait()
    cp_d.wait()

    @plsc.parallel_loop(0, EPW // L, unroll=6)
    def _(i):
        s = src_v[pl.ds(i * L, L)]
        d = dst_v[pl.ds(i * L, L)]
        for j in range(D_HID):
            v = plsc.load_gather(g_v, [s + (j * N)])
            plsc.addupdate_scatter(acc_v, [d + (j * N)], v)

    pltpu.sync_copy(acc_v, out_hbm.at[wid])


def _tc_mm_body(x_ref, w1_ref, h1_ref):
    h1_ref[:] = lax.dot_general(
        w1_ref[:], x_ref[:],
        dimension_numbers=(((0,), (1,)), ((), ())),
        preferred_element_type=jnp.float32,
    )


def _flatten_rows(a):
    return jnp.concatenate([a[j:j + 1, :] for j in range(D_HID)], axis=1)


def _unflatten_rows(a):
    return jnp.concatenate(
        [a[:, j * N:(j + 1) * N] for j in range(D_HID)], axis=0
    )


def _tc1_body(h1_ref, pd_ref, g1_ref, dinv_ref):
    deg = jnp.sum(pd_ref[:], axis=0, keepdims=True) + 1.0
    dinv = lax.rsqrt(deg)
    g1_ref[:] = _flatten_rows(h1_ref[:] * dinv)
    dinv_ref[:] = dinv


_RB = 16
_RSTEPS = NW // _RB


def _tc2_body(p1_ref, g1_ref, dinv_ref, b1_ref, g2_ref, acc_ref):
    i = pl.program_id(0)
    part = jnp.sum(p1_ref[:], axis=0, keepdims=True)

    @pl.when(i == 0)
    def _():
        acc_ref[:] = part

    @pl.when(i > 0)
    def _():
        acc_ref[:] = acc_ref[:] + part

    @pl.when(i == _RSTEPS - 1)
    def _():
        dinv_rep = jnp.concatenate([dinv_ref[:]] * D_HID, axis=1)
        b1_rep = jnp.concatenate(
            [jnp.full((1, N), b1_ref[j, 0], jnp.float32) for j in range(D_HID)],
            axis=1,
        )
        out1 = dinv_rep * (acc_ref[:] + g1_ref[:]) + b1_rep
        r1 = jnp.maximum(out1, 0.0)
        g2_ref[:] = r1 * dinv_rep


def _tc3_body(p2_ref, g2_ref, dinv_ref, w2_ref, b2_ref, out_ref, acc_ref):
    i = pl.program_id(0)
    part = jnp.sum(p2_ref[:], axis=0, keepdims=True)

    @pl.when(i == 0)
    def _():
        acc_ref[:] = part

    @pl.when(i > 0)
    def _():
        acc_ref[:] = acc_ref[:] + part

    @pl.when(i == _RSTEPS - 1)
    def _():
        dinv_rep = jnp.concatenate([dinv_ref[:]] * D_HID, axis=1)
        agg = _unflatten_rows(dinv_rep * (acc_ref[:] + g2_ref[:]))
        o = lax.dot_general(
            w2_ref[:], agg,
            dimension_numbers=(((0,), (0,)), ((), ())),
            preferred_element_type=jnp.float32,
        )
        o = o + b2_ref[:]
        m = jnp.max(o, axis=0, keepdims=True)
        lse = jnp.log(jnp.sum(jnp.exp(o - m), axis=0, keepdims=True)) + m
        out_ref[:] = o - lse


_tc_mm = pl.pallas_call(
    _tc_mm_body,
    out_shape=jax.ShapeDtypeStruct((D_HID, N), jnp.float32),
)

_tc1 = pl.pallas_call(
    _tc1_body,
    out_shape=(
        jax.ShapeDtypeStruct((1, D_HID * N), jnp.float32),
        jax.ShapeDtypeStruct((1, N), jnp.float32),
    ),
)

_tc2 = pl.pallas_call(
    _tc2_body,
    grid=(_RSTEPS,),
    in_specs=[
        pl.BlockSpec((_RB, D_HID * N), lambda i: (i, 0)),
        pl.BlockSpec((1, D_HID * N), lambda i: (0, 0)),
        pl.BlockSpec((1, N), lambda i: (0, 0)),
        pl.BlockSpec((D_HID, 1), lambda i: (0, 0)),
    ],
    out_specs=pl.BlockSpec((1, D_HID * N), lambda i: (0, 0)),
    out_shape=jax.ShapeDtypeStruct((1, D_HID * N), jnp.float32),
    scratch_shapes=[pltpu.VMEM((1, D_HID * N), jnp.float32)],
)

_tc3 = pl.pallas_call(
    _tc3_body,
    grid=(_RSTEPS,),
    in_specs=[
        pl.BlockSpec((_RB, D_HID * N), lambda i: (i, 0)),
        pl.BlockSpec((1, D_HID * N), lambda i: (0, 0)),
        pl.BlockSpec((1, N), lambda i: (0, 0)),
        pl.BlockSpec((D_HID, D_OUT), lambda i: (0, 0)),
        pl.BlockSpec((D_OUT, 1), lambda i: (0, 0)),
    ],
    out_specs=pl.BlockSpec((D_OUT, N), lambda i: (0, 0)),
    out_shape=jax.ShapeDtypeStruct((D_OUT, N), jnp.float32),
    scratch_shapes=[pltpu.VMEM((1, D_HID * N), jnp.float32)],
)


@jax.jit
def kernel(x, edge_index, W1, b1, W2, b2):
    ei = edge_index.astype(jnp.int32).reshape(-1)

    pd = _deg_kernel(ei)
    h1T = _tc_mm(x, W1)
    g1f, dinv = _tc1(h1T, pd)
    p1 = _scatter_kernel(g1f, ei)
    g2f = _tc2(p1, g1f, dinv, b1.reshape(D_HID, 1))
    p2 = _scatter_kernel(g2f, ei)
    outT = _tc3(p2, g2f, dinv, W2, b2.reshape(D_OUT, 1))
    return outT.T


# speedup vs baseline: 1.0012x; 1.0012x over previous
import functools

import jax
import jax.numpy as jnp
from jax import lax
from jax.experimental import pallas as pl
from jax.experimental.pallas import tpu as pltpu
from jax.experimental.pallas import tpu_sc as plsc

N = 10000
E = 320000
D_IN = 128
D_HID = 5
D_OUT = 40

NC = 2
NS = 16
NW = NC * NS
EPW = E // NW
L = 16

_mesh = plsc.VectorSubcoreMesh(
    core_axis_name="c", subcore_axis_name="s", num_cores=NC, num_subcores=NS
)

_sc_params = pltpu.CompilerParams(needs_layout_passes=False)


def _worker_id():
    return lax.axis_index("s") * NC + lax.axis_index("c")


@functools.partial(
    pl.kernel,
    out_type=jax.ShapeDtypeStruct((NW, N), jnp.float32),
    mesh=_mesh,
    scratch_types=[
        pltpu.VMEM((EPW,), jnp.int32),
        pltpu.VMEM((N,), jnp.float32),
        pltpu.SemaphoreType.DMA,
    ],
    compiler_params=_sc_params,
)
def _deg_kernel(ei_hbm, out_hbm, dst_v, acc_v, sem):
    wid = _worker_id()
    base = wid * EPW
    cp = pltpu.async_copy(ei_hbm.at[pl.ds(E + base, EPW)], dst_v, sem)

    zeros = jnp.zeros((L,), jnp.float32)

    def zbody(i, _):
        acc_v[pl.ds(i * L, L)] = zeros
        return 0

    lax.fori_loop(0, N // L, zbody, 0, unroll=25)
    cp.wait()

    ones = jnp.ones((L,), jnp.float32)

    @plsc.parallel_loop(0, EPW // L, unroll=8)
    def _(i):
        d = dst_v[pl.ds(i * L, L)]
        plsc.addupdate_scatter(acc_v, [d], ones)

    pltpu.sync_copy(acc_v, out_hbm.at[wid])


@functools.partial(
    pl.kernel,
    out_type=jax.ShapeDtypeStruct((NW, D_HID * N), jnp.float32),
    mesh=_mesh,
    scratch_types=[
        pltpu.VMEM((D_HID * N,), jnp.float32),
        pltpu.VMEM((EPW,), jnp.int32),
        pltpu.VMEM((EPW,), jnp.int32),
        pltpu.VMEM((D_HID * N,), jnp.float32),
        pltpu.SemaphoreType.DMA,
        pltpu.SemaphoreType.DMA,
        pltpu.SemaphoreType.DMA,
    ],
    compiler_params=_sc_params,
)
def _scatter_kernel(
    g_hbm, ei_hbm, out_hbm, g_v, src_v, dst_v, acc_v, sem_g, sem_s, sem_d
):
    wid = _worker_id()
    base = wid * EPW
    cp_g = pltpu.async_copy(g_hbm.at[0], g_v, sem_g)
    cp_s = pltpu.async_copy(ei_hbm.at[pl.ds(base, EPW)], src_v, sem_s)
    cp_d = pltpu.async_copy(ei_hbm.at[pl.ds(E + base, EPW)], dst_v, sem_d)

    zeros = jnp.zeros((L,), jnp.float32)

    def zbody(i, _):
        acc_v[pl.ds(i * L, L)] = zeros
        return 0

    lax.fori_loop(0, D_HID * N // L, zbody, 0, unroll=25)
    cp_g.wait()
    cp_s.wait()
    cp_d.wait()

    @plsc.parallel_loop(0, EPW // L, unroll=4)
    def _(i):
        s = src_v[pl.ds(i * L, L)]
        d = dst_v[pl.ds(i * L, L)]
        for j in range(D_HID):
            v = plsc.load_gather(g_v, [s + (j * N)])
            plsc.addupdate_scatter(acc_v, [d + (j * N)], v)

    pltpu.sync_copy(acc_v, out_hbm.at[wid])


def _tc_mm_body(x_ref, w1_ref, h1_ref):
    h1_ref[:] = lax.dot_general(
        w1_ref[:], x_ref[:],
        dimension_numbers=(((0,), (1,)), ((), ())),
        preferred_element_type=jnp.float32,
    )


def _flatten_rows(a):
    return jnp.concatenate([a[j:j + 1, :] for j in range(D_HID)], axis=1)


def _unflatten_rows(a):
    return jnp.concatenate(
        [a[:, j * N:(j + 1) * N] for j in range(D_HID)], axis=0
    )


def _tc1_body(h1_ref, pd_ref, g1_ref, dinv_ref):
    deg = jnp.sum(pd_ref[:], axis=0, keepdims=True) + 1.0
    dinv = lax.rsqrt(deg)
    g1_ref[:] = _flatten_rows(h1_ref[:] * dinv)
    dinv_ref[:] = dinv


_RB = 16
_RSTEPS = NW // _RB


def _tc2_body(p1_ref, g1_ref, dinv_ref, b1_ref, g2_ref, acc_ref):
    i = pl.program_id(0)
    part = jnp.sum(p1_ref[:], axis=0, keepdims=True)

    @pl.when(i == 0)
    def _():
        acc_ref[:] = part

    @pl.when(i > 0)
    def _():
        acc_ref[:] = acc_ref[:] + part

    @pl.when(i == _RSTEPS - 1)
    def _():
        dinv_rep = jnp.concatenate([dinv_ref[:]] * D_HID, axis=1)
        b1_rep = jnp.concatenate(
            [jnp.full((1, N), b1_ref[j, 0], jnp.float32) for j in range(D_HID)],
            axis=1,
        )
        out1 = dinv_rep * (acc_ref[:] + g1_ref[:]) + b1_rep
        r1 = jnp.maximum(out1, 0.0)
        g2_ref[:] = r1 * dinv_rep


def _tc3_body(p2_ref, g2_ref, dinv_ref, w2_ref, b2_ref, out_ref, acc_ref):
    i = pl.program_id(0)
    part = jnp.sum(p2_ref[:], axis=0, keepdims=True)

    @pl.when(i == 0)
    def _():
        acc_ref[:] = part

    @pl.when(i > 0)
    def _():
        acc_ref[:] = acc_ref[:] + part

    @pl.when(i == _RSTEPS - 1)
    def _():
        dinv_rep = jnp.concatenate([dinv_ref[:]] * D_HID, axis=1)
        agg = _unflatten_rows(dinv_rep * (acc_ref[:] + g2_ref[:]))
        o = lax.dot_general(
            w2_ref[:], agg,
            dimension_numbers=(((0,), (0,)), ((), ())),
            preferred_element_type=jnp.float32,
        )
        o = o + b2_ref[:]
        m = jnp.max(o, axis=0, keepdims=True)
        lse = jnp.log(jnp.sum(jnp.exp(o - m), axis=0, keepdims=True)) + m
        out_ref[:] = o - lse


_tc_mm = pl.pallas_call(
    _tc_mm_body,
    out_shape=jax.ShapeDtypeStruct((D_HID, N), jnp.float32),
)

_tc1 = pl.pallas_call(
    _tc1_body,
    out_shape=(
        jax.ShapeDtypeStruct((1, D_HID * N), jnp.float32),
        jax.ShapeDtypeStruct((1, N), jnp.float32),
    ),
)

_tc2 = pl.pallas_call(
    _tc2_body,
    grid=(_RSTEPS,),
    in_specs=[
        pl.BlockSpec((_RB, D_HID * N), lambda i: (i, 0)),
        pl.BlockSpec((1, D_HID * N), lambda i: (0, 0)),
        pl.BlockSpec((1, N), lambda i: (0, 0)),
        pl.BlockSpec((D_HID, 1), lambda i: (0, 0)),
    ],
    out_specs=pl.BlockSpec((1, D_HID * N), lambda i: (0, 0)),
    out_shape=jax.ShapeDtypeStruct((1, D_HID * N), jnp.float32),
    scratch_shapes=[pltpu.VMEM((1, D_HID * N), jnp.float32)],
)

_tc3 = pl.pallas_call(
    _tc3_body,
    grid=(_RSTEPS,),
    in_specs=[
        pl.BlockSpec((_RB, D_HID * N), lambda i: (i, 0)),
        pl.BlockSpec((1, D_HID * N), lambda i: (0, 0)),
        pl.BlockSpec((1, N), lambda i: (0, 0)),
        pl.BlockSpec((D_HID, D_OUT), lambda i: (0, 0)),
        pl.BlockSpec((D_OUT, 1), lambda i: (0, 0)),
    ],
    out_specs=pl.BlockSpec((D_OUT, N), lambda i: (0, 0)),
    out_shape=jax.ShapeDtypeStruct((D_OUT, N), jnp.float32),
    scratch_shapes=[pltpu.VMEM((1, D_HID * N), jnp.float32)],
)


@jax.jit
def kernel(x, edge_index, W1, b1, W2, b2):
    ei = edge_index.astype(jnp.int32).reshape(-1)

    pd = _deg_kernel(ei)
    h1T = _tc_mm(x, W1)
    g1f, dinv = _tc1(h1T, pd)
    p1 = _scatter_kernel(g1f, ei)
    g2f = _tc2(p1, g1f, dinv, b1.reshape(D_HID, 1))
    p2 = _scatter_kernel(g2f, ei)
    outT = _tc3(p2, g2f, dinv, W2, b2.reshape(D_OUT, 1))
    return outT.T
